# baseline (speedup 1.0000x reference)
import functools

import jax
import jax.numpy as jnp
from jax import lax
from jax.experimental import pallas as pl
from jax.experimental.pallas import tpu as pltpu
from jax.experimental.pallas import tpu_sc as plsc

B, N, C = 8, 20000, 91
S = 2000
NC, NS, L = 2, 16, 16
WORKERS = 25
CHUNK = S // WORKERS
OUT_COLS = 7


def _sc_pick(boxes_flat, scores_flat, sel_flat):
    mesh = plsc.VectorSubcoreMesh(core_axis_name="c", subcore_axis_name="s")

    @functools.partial(
        pl.kernel,
        mesh=mesh,
        out_type=jax.ShapeDtypeStruct((S * OUT_COLS,), jnp.float32),
        scratch_types=[
            pltpu.VMEM((CHUNK,), jnp.int32),
            pltpu.VMEM((CHUNK,), jnp.int32),
            pltpu.VMEM((CHUNK,), jnp.int32),
            pltpu.VMEM((CHUNK,), jnp.int32),
            pltpu.VMEM((CHUNK,), jnp.int32),
            pltpu.VMEM((CHUNK,), jnp.int32),
            pltpu.VMEM((CHUNK,), jnp.int32),
            pltpu.VMEM((CHUNK,), jnp.int32),
            pltpu.VMEM((CHUNK,), jnp.int32),
            pltpu.VMEM((CHUNK,), jnp.int32),
            pltpu.VMEM((CHUNK,), jnp.int32),
            pltpu.VMEM((CHUNK,), jnp.int32),
            pltpu.VMEM((CHUNK,), jnp.int32),
            pltpu.VMEM((CHUNK,), jnp.int32),
            pltpu.VMEM((CHUNK,), jnp.int32),
            pltpu.VMEM((CHUNK,), jnp.int32),
            pltpu.VMEM((CHUNK,), jnp.int32),
            pltpu.VMEM((CHUNK,), jnp.int32),
            pltpu.VMEM((CHUNK,), jnp.float32),
            pltpu.VMEM((CHUNK,), jnp.float32),
            pltpu.VMEM((CHUNK,), jnp.float32),
            pltpu.VMEM((CHUNK,), jnp.float32),
            pltpu.VMEM((CHUNK,), jnp.float32),
            pltpu.VMEM((CHUNK,), jnp.float32),
            pltpu.VMEM((CHUNK,), jnp.float32),
            pltpu.SemaphoreType.DMA,
            pltpu.SemaphoreType.DMA,
            pltpu.SemaphoreType.DMA,
        ],
    )
    def k(boxes_hbm, scores_hbm, sel_hbm, out_hbm,
          gb_v, gl_v, gn_v, b_v, l_v, n_v,
          c0i_v, c1i_v, c2i_v, c3i_v, si_v,
          o0i_v, o1i_v, o2i_v, o3i_v, o4i_v, o5i_v, o6i_v,
          bf_v, c0_v, c1_v, c2_v, c3_v, sc_v, lf_v,
          sem_a, sem_b, sem_c):
        wid = lax.axis_index("s") * NC + lax.axis_index("c")

        @pl.when(wid < WORKERS)
        def _():
            base = wid * CHUNK
            iota = lax.iota(jnp.int32, L)
            for j in range(CHUNK // L):
                sl = pl.ds(j * L, L)
                s3 = (iota + (base + j * L)) * 3
                gb_v[sl] = s3
                gl_v[sl] = s3 + 1
                gn_v[sl] = s3 + 2
            cps = [pltpu.async_copy(sel_hbm.at[gb_v], b_v, sem_a),
                   pltpu.async_copy(sel_hbm.at[gl_v], l_v, sem_a),
                   pltpu.async_copy(sel_hbm.at[gn_v], n_v, sem_a)]
            for cp in cps:
                cp.wait()

            for j in range(CHUNK // L):
                sl = pl.ds(j * L, L)
                row = b_v[sl] * N + n_v[sl]
                row4 = row * 4
                c0i_v[sl] = row4
                c1i_v[sl] = row4 + 1
                c2i_v[sl] = row4 + 2
                c3i_v[sl] = row4 + 3
                si_v[sl] = row * C + l_v[sl]
                o = (iota + (base + j * L)) * OUT_COLS
                o0i_v[sl] = o
                o1i_v[sl] = o + 1
                o2i_v[sl] = o + 2
                o3i_v[sl] = o + 3
                o4i_v[sl] = o + 4
                o5i_v[sl] = o + 5
                o6i_v[sl] = o + 6
                bf_v[sl] = b_v[sl].astype(jnp.float32)
                lf_v[sl] = l_v[sl].astype(jnp.float32)

            cps = [pltpu.async_copy(boxes_hbm.at[c0i_v], c0_v, sem_b),
                   pltpu.async_copy(boxes_hbm.at[c1i_v], c1_v, sem_b),
                   pltpu.async_copy(boxes_hbm.at[c2i_v], c2_v, sem_b),
                   pltpu.async_copy(boxes_hbm.at[c3i_v], c3_v, sem_b),
                   pltpu.async_copy(scores_hbm.at[si_v], sc_v, sem_b)]
            for cp in cps:
                cp.wait()

            cps = [pltpu.async_copy(bf_v, out_hbm.at[o0i_v], sem_c),
                   pltpu.async_copy(c0_v, out_hbm.at[o1i_v], sem_c),
                   pltpu.async_copy(c1_v, out_hbm.at[o2i_v], sem_c),
                   pltpu.async_copy(c2_v, out_hbm.at[o3i_v], sem_c),
                   pltpu.async_copy(c3_v, out_hbm.at[o4i_v], sem_c),
                   pltpu.async_copy(sc_v, out_hbm.at[o5i_v], sem_c),
                   pltpu.async_copy(lf_v, out_hbm.at[o6i_v], sem_c)]
            for cp in cps:
                cp.wait()

    return k(boxes_flat, scores_flat, sel_flat)


def kernel(pred_boxes, pred_scores, selected_indexes):
    sel_flat = selected_indexes.astype(jnp.int32).reshape(S * 3)
    boxes_flat = pred_boxes.reshape(B * N * 4)
    scores_flat = pred_scores.reshape(B * N * C)
    out = _sc_pick(boxes_flat, scores_flat, sel_flat)
    return out.reshape(S, OUT_COLS)


# speedup vs baseline: 2.1446x; 2.1446x over previous
import functools

import jax
import jax.numpy as jnp
from jax import lax
from jax.experimental import pallas as pl
from jax.experimental.pallas import tpu as pltpu
from jax.experimental.pallas import tpu_sc as plsc

B, N, C = 8, 20000, 91
S = 2000
NC, NS, L = 2, 16, 16
WORKERS = 25
CHUNK = S // WORKERS


def _sc_pick(boxes1d, scores3d, bidx, lidx, nidx):
    mesh = plsc.VectorSubcoreMesh(core_axis_name="c", subcore_axis_name="s")

    @functools.partial(
        pl.kernel,
        mesh=mesh,
        compiler_params=pltpu.CompilerParams(needs_layout_passes=False),
        out_type=[jax.ShapeDtypeStruct((S,), jnp.float32) for _ in range(7)],
        scratch_types=[
            pltpu.VMEM((CHUNK,), jnp.int32),
            pltpu.VMEM((CHUNK,), jnp.int32),
            pltpu.VMEM((CHUNK,), jnp.int32),
            pltpu.VMEM((CHUNK,), jnp.int32),
            pltpu.VMEM((CHUNK,), jnp.int32),
            pltpu.VMEM((CHUNK,), jnp.int32),
            pltpu.VMEM((CHUNK,), jnp.int32),
            pltpu.VMEM((CHUNK,), jnp.float32),
            pltpu.VMEM((CHUNK,), jnp.float32),
            pltpu.VMEM((CHUNK,), jnp.float32),
            pltpu.VMEM((CHUNK,), jnp.float32),
            pltpu.VMEM((CHUNK, C), jnp.float32),
            pltpu.VMEM((CHUNK,), jnp.float32),
            pltpu.VMEM((CHUNK,), jnp.float32),
            pltpu.VMEM((CHUNK,), jnp.float32),
            pltpu.SemaphoreType.DMA,
            pltpu.SemaphoreType.DMA,
        ],
    )
    def k(boxes_hbm, scores_hbm, bidx_hbm, lidx_hbm, nidx_hbm,
          o0_hbm, o1_hbm, o2_hbm, o3_hbm, o4_hbm, o5_hbm, o6_hbm,
          b_v, l_v, n_v, c0i_v, c1i_v, c2i_v, c3i_v,
          c0_v, c1_v, c2_v, c3_v, rows_v, sc_v, bf_v, lf_v,
          sem_a, sem_b):
        wid = lax.axis_index("s") * NC + lax.axis_index("c")

        @pl.when(wid < WORKERS)
        def _():
            base = wid * CHUNK
            cps = [pltpu.async_copy(bidx_hbm.at[pl.ds(base, CHUNK)], b_v,
                                    sem_a),
                   pltpu.async_copy(lidx_hbm.at[pl.ds(base, CHUNK)], l_v,
                                    sem_a),
                   pltpu.async_copy(nidx_hbm.at[pl.ds(base, CHUNK)], n_v,
                                    sem_a),
                   ]
            for cp in cps:
                cp.wait()

            for j in range(CHUNK // L):
                sl = pl.ds(j * L, L)
                row4 = (b_v[sl] * N + n_v[sl]) * 4
                c0i_v[sl] = row4
                c1i_v[sl] = row4 + 1
                c2i_v[sl] = row4 + 2
                c3i_v[sl] = row4 + 3
                bf_v[sl] = b_v[sl].astype(jnp.float32)
                lf_v[sl] = l_v[sl].astype(jnp.float32)

            box_cps = [pltpu.async_copy(boxes_hbm.at[c0i_v], c0_v, sem_b),
                       pltpu.async_copy(boxes_hbm.at[c1i_v], c1_v, sem_b),
                       pltpu.async_copy(boxes_hbm.at[c2i_v], c2_v, sem_b),
                       pltpu.async_copy(boxes_hbm.at[c3i_v], c3_v, sem_b)]

            row_cps = []
            for j in range(CHUNK // L):
                bvec = b_v[pl.ds(j * L, L)]
                nvec = n_v[pl.ds(j * L, L)]
                for kk in range(L):
                    i = j * L + kk
                    row_cps.append(pltpu.async_copy(
                        scores_hbm.at[bvec[kk], pl.ds(nvec[kk], 1)],
                        rows_v.at[pl.ds(i, 1)], sem_a))
            for cp in row_cps:
                cp.wait()

            iota = lax.iota(jnp.int32, L)
            for j in range(CHUNK // L):
                sl = pl.ds(j * L, L)
                rid = iota + (j * L)
                sc_v[sl] = plsc.load_gather(rows_v, [rid, l_v[sl]])

            for cp in box_cps:
                cp.wait()

            outs = [(bf_v, o0_hbm), (c0_v, o1_hbm), (c1_v, o2_hbm),
                    (c2_v, o3_hbm), (c3_v, o4_hbm), (sc_v, o5_hbm),
                    (lf_v, o6_hbm)]
            cps = [pltpu.async_copy(src, dst.at[pl.ds(base, CHUNK)], sem_b)
                   for src, dst in outs]
            for cp in cps:
                cp.wait()

    return k(boxes1d, scores3d, bidx, lidx, nidx)


def kernel(pred_boxes, pred_scores, selected_indexes):
    sel = selected_indexes.astype(jnp.int32)
    boxes1d = pred_boxes.reshape(B * N * 4)
    cols = _sc_pick(boxes1d, pred_scores, sel[:, 0], sel[:, 1], sel[:, 2])
    return jnp.stack(cols, axis=1)


# speedup vs baseline: 3.2254x; 1.5040x over previous
import functools

import jax
import jax.numpy as jnp
from jax import lax
from jax.experimental import pallas as pl
from jax.experimental.pallas import tpu as pltpu
from jax.experimental.pallas import tpu_sc as plsc

B, N, C = 8, 20000, 91
S = 2000
NC, NS, L = 2, 16, 16
WORKERS = 25
CHUNK = S // WORKERS


def _sc_pick(boxes3d, scores3d, bidx, lidx, nidx):
    mesh = plsc.VectorSubcoreMesh(core_axis_name="c", subcore_axis_name="s")

    @functools.partial(
        pl.kernel,
        mesh=mesh,
        compiler_params=pltpu.CompilerParams(needs_layout_passes=False),
        out_type=[jax.ShapeDtypeStruct((S,), jnp.float32) for _ in range(7)],
        scratch_types=[
            pltpu.VMEM((CHUNK,), jnp.int32),
            pltpu.VMEM((CHUNK,), jnp.int32),
            pltpu.VMEM((CHUNK,), jnp.int32),
            pltpu.VMEM((CHUNK, 4), jnp.float32),
            pltpu.VMEM((CHUNK,), jnp.float32),
            pltpu.VMEM((CHUNK,), jnp.float32),
            pltpu.VMEM((CHUNK,), jnp.float32),
            pltpu.VMEM((CHUNK,), jnp.float32),
            pltpu.VMEM((CHUNK, C), jnp.float32),
            pltpu.VMEM((CHUNK,), jnp.float32),
            pltpu.VMEM((CHUNK,), jnp.float32),
            pltpu.VMEM((CHUNK,), jnp.float32),
            pltpu.SemaphoreType.DMA,
            pltpu.SemaphoreType.DMA,
        ],
    )
    def k(boxes_hbm, scores_hbm, bidx_hbm, lidx_hbm, nidx_hbm,
          o0_hbm, o1_hbm, o2_hbm, o3_hbm, o4_hbm, o5_hbm, o6_hbm,
          b_v, l_v, n_v, boxrows_v, c0_v, c1_v, c2_v, c3_v,
          rows_v, sc_v, bf_v, lf_v, sem_a, sem_b):
        wid = lax.axis_index("s") * NC + lax.axis_index("c")

        @pl.when(wid < WORKERS)
        def _():
            base = wid * CHUNK
            cps = [pltpu.async_copy(bidx_hbm.at[pl.ds(base, CHUNK)], b_v,
                                    sem_a),
                   pltpu.async_copy(lidx_hbm.at[pl.ds(base, CHUNK)], l_v,
                                    sem_a),
                   pltpu.async_copy(nidx_hbm.at[pl.ds(base, CHUNK)], n_v,
                                    sem_a)]
            for cp in cps:
                cp.wait()

            row_cps = []
            for j in range(CHUNK // L):
                bvec = b_v[pl.ds(j * L, L)]
                nvec = n_v[pl.ds(j * L, L)]
                for kk in range(L):
                    i = j * L + kk
                    row_cps.append(pltpu.async_copy(
                        boxes_hbm.at[bvec[kk], pl.ds(nvec[kk], 1)],
                        boxrows_v.at[pl.ds(i, 1)], sem_b))
                    row_cps.append(pltpu.async_copy(
                        scores_hbm.at[bvec[kk], pl.ds(nvec[kk], 1)],
                        rows_v.at[pl.ds(i, 1)], sem_a))

            for j in range(CHUNK // L):
                sl = pl.ds(j * L, L)
                bf_v[sl] = b_v[sl].astype(jnp.float32)
                lf_v[sl] = l_v[sl].astype(jnp.float32)

            for cp in row_cps:
                cp.wait()

            iota = lax.iota(jnp.int32, L)
            for j in range(CHUNK // L):
                sl = pl.ds(j * L, L)
                rid = iota + (j * L)
                sc_v[sl] = plsc.load_gather(rows_v, [rid, l_v[sl]])
                for c, cv in enumerate((c0_v, c1_v, c2_v, c3_v)):
                    cv[sl] = plsc.load_gather(
                        boxrows_v, [rid, jnp.full((L,), c, jnp.int32)])

            outs = [(bf_v, o0_hbm), (c0_v, o1_hbm), (c1_v, o2_hbm),
                    (c2_v, o3_hbm), (c3_v, o4_hbm), (sc_v, o5_hbm),
                    (lf_v, o6_hbm)]
            cps = [pltpu.async_copy(src, dst.at[pl.ds(base, CHUNK)], sem_b)
                   for src, dst in outs]
            for cp in cps:
                cp.wait()

    return k(boxes3d, scores3d, bidx, lidx, nidx)


def kernel(pred_boxes, pred_scores, selected_indexes):
    sel = selected_indexes.astype(jnp.int32)
    cols = _sc_pick(pred_boxes, pred_scores, sel[:, 0], sel[:, 1], sel[:, 2])
    return jnp.stack(cols, axis=1)
